# baseline (speedup 1.0000x reference)
import jax
import jax.numpy as jnp
from jax import lax
from jax.experimental import pallas as pl
from jax.experimental.pallas import tpu as pltpu
from jax.experimental.pallas import tpu_sc as plsc

_NUM_CORES = 2
_NUM_SUBCORES = 16
_NUM_WORKERS = _NUM_CORES * _NUM_SUBCORES


def _make_sc_broadcast(batch: int, t: int, embed: int, table_rows: int):
    assert batch % _NUM_WORKERS == 0
    b_per_w = batch // _NUM_WORKERS

    mesh = plsc.VectorSubcoreMesh(
        core_axis_name="c", subcore_axis_name="s"
    )

    @pl.kernel(
        out_type=jax.ShapeDtypeStruct((batch, t, embed), jnp.float32),
        mesh=mesh,
        compiler_params=pltpu.CompilerParams(use_tc_tiling_on_sc=False),
        scratch_types=[
            pltpu.VMEM((t,), jnp.int32),
            pltpu.VMEM((t, embed), jnp.float32),
            pltpu.SemaphoreType.DMA,
            pltpu.SemaphoreType.DMA,
        ],
    )
    def sc_kernel(idx_hbm, table_hbm, out_hbm, idx_v, rows_v, gsem, wsem):
        wid = lax.axis_index("s") * _NUM_CORES + lax.axis_index("c")
        base = wid * b_per_w

        pltpu.sync_copy(idx_hbm, idx_v)
        pltpu.async_copy(table_hbm.at[idx_v], rows_v, gsem).wait()

        def fire(i, carry):
            pltpu.async_copy(rows_v, out_hbm.at[base + i], wsem)
            return carry

        lax.fori_loop(0, b_per_w, fire, 0, unroll=False)

        def drain(i, carry):
            pltpu.make_async_copy(rows_v, out_hbm.at[base + i], wsem).wait()
            return carry

        lax.fori_loop(0, b_per_w, drain, 0, unroll=False)

    return sc_kernel


def kernel(x, x_index, table):
    batch, t = x.shape
    table_rows, embed = table.shape
    sc = _make_sc_broadcast(batch, t, embed, table_rows)
    return sc(x_index.astype(jnp.int32), table)


# speedup vs baseline: 1.3370x; 1.3370x over previous
import functools

import jax
import jax.numpy as jnp
from jax import lax
from jax.experimental import pallas as pl
from jax.experimental.pallas import tpu as pltpu
from jax.experimental.pallas import tpu_sc as plsc

_NUM_CORES = 2


@functools.cache
def _sc_gather(t: int, embed: int):
    mesh = plsc.VectorSubcoreMesh(core_axis_name="c", subcore_axis_name="s")

    @pl.kernel(
        out_type=jax.ShapeDtypeStruct((t, embed), jnp.float32),
        mesh=mesh,
        compiler_params=pltpu.CompilerParams(use_tc_tiling_on_sc=False),
        scratch_types=[
            pltpu.VMEM((t,), jnp.int32),
            pltpu.VMEM((t, embed), jnp.float32),
            pltpu.SemaphoreType.DMA,
        ],
    )
    def gather_kernel(idx_hbm, table_hbm, out_hbm, idx_v, rows_v, gsem):
        wid = lax.axis_index("s") * _NUM_CORES + lax.axis_index("c")

        @pl.when(wid == 0)
        def _():
            pltpu.sync_copy(idx_hbm, idx_v)
            pltpu.async_copy(table_hbm.at[idx_v], rows_v, gsem).wait()
            pltpu.sync_copy(rows_v, out_hbm)

    return gather_kernel


@functools.cache
def _tc_broadcast(batch: int, t: int, embed: int, bblk: int):
    def body(emb_ref, out_ref):
        out_ref[...] = jnp.broadcast_to(
            emb_ref[...][None], (bblk, t, embed)
        )

    return pl.pallas_call(
        body,
        grid=(batch // bblk,),
        in_specs=[pl.BlockSpec((t, embed), lambda i: (0, 0))],
        out_specs=pl.BlockSpec((bblk, t, embed), lambda i: (i, 0, 0)),
        out_shape=jax.ShapeDtypeStruct((batch, t, embed), jnp.float32),
    )


def kernel(x, x_index, table):
    batch, t = x.shape
    _, embed = table.shape
    emb = _sc_gather(t, embed)(x_index.astype(jnp.int32), table)
    return _tc_broadcast(batch, t, embed, 128)(emb)


# speedup vs baseline: 1.4563x; 1.0892x over previous
import functools

import jax
import jax.numpy as jnp
from jax import lax
from jax.experimental import pallas as pl
from jax.experimental.pallas import tpu as pltpu


@functools.cache
def _tc_bcast(batch: int, t: int, rows: int, embed: int, bblk: int):
    def body(idx_ref, table_ref, out_ref, emb_ref):
        @pl.when(pl.program_id(0) == 0)
        def _():
            onehot = (
                idx_ref[...][:, None]
                == lax.broadcasted_iota(jnp.int32, (t, rows), 1)
            ).astype(jnp.float32)
            emb_ref[...] = jnp.dot(
                onehot, table_ref[...], preferred_element_type=jnp.float32
            )

        out_ref[...] = jnp.broadcast_to(emb_ref[...][None], (bblk, t, embed))

    return pl.pallas_call(
        body,
        grid=(batch // bblk,),
        in_specs=[
            pl.BlockSpec((t,), lambda i: (0,)),
            pl.BlockSpec((rows, embed), lambda i: (0, 0)),
        ],
        out_specs=pl.BlockSpec((bblk, t, embed), lambda i: (i, 0, 0)),
        out_shape=jax.ShapeDtypeStruct((batch, t, embed), jnp.float32),
        scratch_shapes=[pltpu.VMEM((t, embed), jnp.float32)],
    )


def kernel(x, x_index, table):
    batch, t = x.shape
    rows, embed = table.shape
    return _tc_bcast(batch, t, rows, embed, 128)(
        x_index.astype(jnp.int32), table
    )


# speedup vs baseline: 9.1281x; 6.2682x over previous
import functools

import jax
import jax.numpy as jnp
from jax import lax
from jax.experimental import pallas as pl
from jax.experimental.pallas import tpu as pltpu


@functools.cache
def _tc_bcast(batch: int, t: int, rows: int, embed: int, tblk: int):
    def body(idx_ref, table_ref, out_ref, emb_ref):
        @pl.when(pl.program_id(0) == 0)
        def _():
            onehot = (
                idx_ref[...][:, None]
                == lax.broadcasted_iota(jnp.int32, (t, rows), 1)
            ).astype(jnp.float32)
            emb_ref[...] = jnp.dot(
                onehot, table_ref[...], preferred_element_type=jnp.float32
            )

        i = pl.program_id(0)
        blk = emb_ref[pl.ds(i * tblk, tblk), :]
        out_ref[...] = jnp.broadcast_to(blk[:, :, None], (tblk, embed, batch))

    return pl.pallas_call(
        body,
        grid=(t // tblk,),
        in_specs=[
            pl.BlockSpec((t,), lambda i: (0,)),
            pl.BlockSpec((rows, embed), lambda i: (0, 0)),
        ],
        out_specs=pl.BlockSpec((tblk, embed, batch), lambda i: (i, 0, 0)),
        out_shape=jax.ShapeDtypeStruct((t, embed, batch), jnp.float32),
        scratch_shapes=[pltpu.VMEM((t, embed), jnp.float32)],
    )


def kernel(x, x_index, table):
    batch, t = x.shape
    rows, embed = table.shape
    out_teb = _tc_bcast(batch, t, rows, embed, 4)(
        x_index.astype(jnp.int32), table
    )
    return jnp.transpose(out_teb, (2, 0, 1))
